# baseline (speedup 1.0000x reference)
import jax
import jax.numpy as jnp
from jax.experimental import pallas as pl
from jax.experimental.pallas import tpu as pltpu

K = 3
ALPHA = 8.3

_CT = 128
_HC = 16
_G = 4


def _shift_h(x, o):
    if o == 0:
        return x
    z = jnp.zeros_like(x[..., :1, :])
    if o == 1:
        return jnp.concatenate([x[..., 1:, :], z], axis=-2)
    return jnp.concatenate([z, x[..., :-1, :]], axis=-2)


def _shift_w(x, o):
    if o == 0:
        return x
    z = jnp.zeros_like(x[..., :, :1])
    if o == 1:
        return jnp.concatenate([x[..., :, 1:], z], axis=-1)
    return jnp.concatenate([z, x[..., :, :-1]], axis=-1)


def _body(depth_ref, img_a_ref, img_b_ref, out_ref, wn_ref):
    c_idx = pl.program_id(1)
    H = out_ref.shape[2]
    W = out_ref.shape[3]

    @pl.when(c_idx == 0)
    def _():
        d = depth_ref[0, 0]
        ws = []
        for oi in (-1, 0, 1):
            dh = _shift_h(d, oi)
            for oj in (-1, 0, 1):
                dk = _shift_w(dh, oj)
                ws.append(jnp.exp(-ALPHA * jnp.abs(d - dk)))
        den = ws[0]
        for w in ws[1:]:
            den = den + w
        inv = 1.0 / den
        k = 0
        for oi in (-1, 0, 1):
            for oj in (-1, 0, 1):
                wn_ref[k] = _shift_w(ws[k] * inv, -oj)
                k += 1

    zrow = jnp.zeros((1, W), jnp.float32)

    def _xh(c, h0):
        img_ref = img_a_ref if c < _CT // 2 else img_b_ref
        cl = c % (_CT // 2)
        xh = {}
        for oi in (-1, 0, 1):
            s = h0 + oi
            if s < 0:
                xh[oi] = jnp.concatenate(
                    [zrow, img_ref[0, cl, 0:_HC - 1, :]], axis=0)
            elif s + _HC > H:
                xh[oi] = jnp.concatenate(
                    [img_ref[0, cl, s:H, :], zrow], axis=0)
            else:
                xh[oi] = img_ref[0, cl, s:s + _HC, :]
        return xh

    for c0 in range(0, _CT, _G):
        for h0 in range(0, H, _HC):
            xhs = [_xh(c0 + g, h0) for g in range(_G)]
            yss = [[None] * K for _ in range(_G)]
            for j_idx in range(K):
                for i_idx, di in enumerate((-1, 0, 1)):
                    w = wn_ref[i_idx * K + j_idx, h0:h0 + _HC, :]
                    for g in range(_G):
                        t = w * xhs[g][di]
                        y = yss[g][j_idx]
                        yss[g][j_idx] = t if y is None else y + t
            for g in range(_G):
                ys = yss[g]
                acc = (pltpu.roll(ys[0], 1, axis=1) + ys[1]
                       + pltpu.roll(ys[2], W - 1, axis=1))
                out_ref[0, c0 + g, h0:h0 + _HC, :] = acc


def kernel(img, depth):
    B, C, H, W = img.shape
    return pl.pallas_call(
        _body,
        out_shape=jax.ShapeDtypeStruct((B, C, H, W), img.dtype),
        grid=(B, C // _CT),
        in_specs=[
            pl.BlockSpec((1, 1, H, W), lambda b, c: (b, 0, 0, 0)),
            pl.BlockSpec((1, _CT // 2, H, W), lambda b, c: (b, 2 * c, 0, 0)),
            pl.BlockSpec((1, _CT // 2, H, W),
                         lambda b, c: (b, 2 * c + 1, 0, 0)),
        ],
        out_specs=pl.BlockSpec((1, _CT, H, W), lambda b, c: (b, c, 0, 0)),
        scratch_shapes=[pltpu.VMEM((K * K, H, W), jnp.float32)],
        compiler_params=pltpu.CompilerParams(
            dimension_semantics=("parallel", "arbitrary"),
            vmem_limit_bytes=56 * 1024 * 1024,
        ),
        name="depth_avg_pool",
    )(depth, img, img)
